# Initial kernel scaffold; baseline (speedup 1.0000x reference)
import jax
import jax.numpy as jnp
from jax.experimental import pallas as pl


def kernel(x, table):
    raise NotImplementedError("write your pallas kernel here")


# speedup vs baseline: 2.2760x; 2.2760x over previous
import functools

import jax
import jax.numpy as jnp
from jax import lax
from jax.experimental import pallas as pl
from jax.experimental.pallas import tpu as pltpu
from jax.experimental.pallas import tpu_sc as plsc

_S = 8192
_D = 2048
_B = 4
_NC = 2
_NS = 16
_NW = _NC * _NS
_ROWS_PER_W = _S // _NW
_CH = 16
_NCHUNK = _ROWS_PER_W // _CH

_mesh = plsc.VectorSubcoreMesh(core_axis_name="c", subcore_axis_name="s")


@functools.partial(
    pl.kernel,
    mesh=_mesh,
    out_type=jax.ShapeDtypeStruct((_B * _S, _D), jnp.float32),
    scratch_types=[
        pltpu.VMEM((_CH, _D), jnp.float32),
        pltpu.SemaphoreType.DMA,
        pltpu.SemaphoreType.DMA,
    ],
)
def _bcast_rows(table_hbm, out_hbm, buf, rsem, wsem):
    wid = lax.axis_index("s") * _NC + lax.axis_index("c")
    base = wid * _ROWS_PER_W

    def body(i, carry):
        r = base + i * _CH
        pltpu.async_copy(table_hbm.at[pl.ds(r, _CH)], buf, rsem).wait()
        copies = [
            pltpu.async_copy(buf, out_hbm.at[pl.ds(b * _S + r, _CH)], wsem)
            for b in range(_B)
        ]
        for c in copies:
            c.wait()
        return carry

    lax.fori_loop(0, _NCHUNK, body, 0)


def kernel(x, table):
    del x
    out = _bcast_rows(table)
    return out.reshape(_B, _S, _D)


# speedup vs baseline: 2.3349x; 1.0259x over previous
import functools

import jax
import jax.numpy as jnp
from jax import lax
from jax.experimental import pallas as pl
from jax.experimental.pallas import tpu as pltpu
from jax.experimental.pallas import tpu_sc as plsc

_S = 8192
_D = 2048
_B = 4
_NC = 2
_NS = 16
_NW = _NC * _NS
_ROWS_PER_W = _S // _NW
_CH = 16
_NCHUNK = _ROWS_PER_W // _CH

_mesh = plsc.VectorSubcoreMesh(core_axis_name="c", subcore_axis_name="s")


_NBUF = 2


@functools.partial(
    pl.kernel,
    mesh=_mesh,
    out_type=jax.ShapeDtypeStruct((_B * _S, _D), jnp.float32),
    scratch_types=[
        pltpu.VMEM((_NBUF, _CH, _D), jnp.float32),
        pltpu.SemaphoreType.DMA,
        pltpu.SemaphoreType.DMA,
    ],
)
def _bcast_rows(table_hbm, out_hbm, buf, rsem, wsem):
    wid = lax.axis_index("s") * _NC + lax.axis_index("c")
    base = wid * _ROWS_PER_W

    def issue_read(i):
        return pltpu.async_copy(
            table_hbm.at[pl.ds(base + i * _CH, _CH)], buf.at[i % _NBUF], rsem
        )

    def issue_writes(i):
        return [
            pltpu.async_copy(
                buf.at[i % _NBUF], out_hbm.at[pl.ds(b * _S + base + i * _CH, _CH)], wsem
            )
            for b in range(_B)
        ]

    rh = [None] * _NCHUNK
    wh = [None] * _NCHUNK
    rh[0] = issue_read(0)
    rh[1] = issue_read(1)
    rh[0].wait()
    wh[0] = issue_writes(0)
    for i in range(1, _NCHUNK):
        rh[i].wait()
        wh[i] = issue_writes(i)
        if i + 1 < _NCHUNK:
            for c in wh[i - 1]:
                c.wait()
            rh[i + 1] = issue_read(i + 1)
    for c in wh[_NCHUNK - 2] + wh[_NCHUNK - 1]:
        c.wait()


def kernel(x, table):
    del x
    out = _bcast_rows(table)
    return out.reshape(_B, _S, _D)
